# baseline (speedup 1.0000x reference)
import numpy as np
import jax
import jax.numpy as jnp
from jax.experimental import pallas as pl
from jax.experimental.pallas import tpu as pltpu

_NUM_ROWS = 100000
_D = 256
_ORD_LEN = 64
_OFFSETS = (0, 137, 271)
_STRIDE = 7919

_IDX = [((np.arange(_ORD_LEN, dtype=np.int64) * _STRIDE + off) % _NUM_ROWS)
        .astype(int).tolist() for off in _OFFSETS]


_NSEM = 8


def _tc_body(l0, l1, l2, out_ref, sems):
    n = 0
    for l, ref in enumerate((l0, l1, l2)):
        for i, row in enumerate(_IDX[l]):
            pltpu.make_async_copy(
                ref.at[pl.ds(row, 1)],
                out_ref.at[pl.ds(l * _ORD_LEN + i, 1)],
                sems.at[n % _NSEM]).start()
            n += 1
    rows_per_sem = len(_OFFSETS) * _ORD_LEN // _NSEM
    for s in range(_NSEM):
        pltpu.make_async_copy(l0.at[pl.ds(0, rows_per_sem)],
                              out_ref.at[pl.ds(0, rows_per_sem)],
                              sems.at[s]).wait()


def kernel(layer_0, layer_1, layer_2):
    return pl.pallas_call(
        _tc_body,
        out_shape=jax.ShapeDtypeStruct((len(_OFFSETS) * _ORD_LEN, _D),
                                       jnp.float32),
        in_specs=[pl.BlockSpec(memory_space=pltpu.MemorySpace.HBM)] * 3,
        out_specs=pl.BlockSpec((len(_OFFSETS) * _ORD_LEN, _D),
                               lambda: (0, 0)),
        scratch_shapes=[pltpu.SemaphoreType.DMA((_NSEM,))],
    )(layer_0, layer_1, layer_2)


# speedup vs baseline: 1.0272x; 1.0272x over previous
import numpy as np
import jax
import jax.numpy as jnp
from jax.experimental import pallas as pl
from jax.experimental.pallas import tpu as pltpu

_NUM_ROWS = 100000
_D = 256
_ORD_LEN = 64
_OFFSETS = (0, 137, 271)
_STRIDE = 7919

_IDX = [((np.arange(_ORD_LEN, dtype=np.int64) * _STRIDE + off) % _NUM_ROWS)
        .astype(int).tolist() for off in _OFFSETS]


def _tc_body(l0, l1, l2, out_ref, sem):
    for l, ref in enumerate((l0, l1, l2)):
        for i, row in enumerate(_IDX[l]):
            pltpu.make_async_copy(
                ref.at[pl.ds(row, 1)],
                out_ref.at[pl.ds(l * _ORD_LEN + i, 1)],
                sem).start()
    pltpu.make_async_copy(l0.at[pl.ds(0, len(_OFFSETS) * _ORD_LEN)],
                          out_ref, sem).wait()


def kernel(layer_0, layer_1, layer_2):
    return pl.pallas_call(
        _tc_body,
        out_shape=jax.ShapeDtypeStruct((len(_OFFSETS) * _ORD_LEN, _D),
                                       jnp.float32),
        in_specs=[pl.BlockSpec(memory_space=pltpu.MemorySpace.HBM)] * 3,
        out_specs=pl.BlockSpec((len(_OFFSETS) * _ORD_LEN, _D),
                               lambda: (0, 0)),
        scratch_shapes=[pltpu.SemaphoreType.DMA],
    )(layer_0, layer_1, layer_2)
